# baseline (speedup 1.0000x reference)
import functools

import jax
import jax.numpy as jnp
from jax import lax
from jax.experimental import pallas as pl
from jax.experimental.pallas import tpu as pltpu
from jax.experimental.pallas import tpu_sc as plsc

BATCH = 4096
SEQ = 50
SEQP = 56
D = 512
NC = 2
NS = 16
NW = NC * NS
NSHARD = 4
BS = BATCH // NSHARD
ROWS_W = BS // NW
NBUF = 4
BLK = 32


def _make_emb():
    mesh = plsc.VectorSubcoreMesh(core_axis_name="c", subcore_axis_name="s")

    @functools.partial(
        pl.kernel,
        mesh=mesh,
        out_type=jax.ShapeDtypeStruct((BS, SEQP, D), jnp.float32),
        scratch_types=[
            pltpu.VMEM((ROWS_W * SEQP,), jnp.int32),
        ]
        + [pltpu.VMEM((SEQP, D), jnp.float32) for _ in range(NBUF)]
        + [pltpu.SemaphoreType.DMA for _ in range(2 * NBUF)],
    )
    def emb(table_hbm, idx_hbm, out_hbm, idx_v, *bufs_and_sems):
        bufs = bufs_and_sems[:NBUF]
        gsem = bufs_and_sems[NBUF : 2 * NBUF]
        ssem = bufs_and_sems[2 * NBUF : 3 * NBUF]

        wid = lax.axis_index("s") * NC + lax.axis_index("c")
        base = wid * ROWS_W
        pltpu.sync_copy(idx_hbm.at[pl.ds(base * SEQP, ROWS_W * SEQP)], idx_v)

        def g_copy(c, b):
            return pltpu.make_async_copy(
                table_hbm.at[idx_v.at[pl.ds(c * SEQP, SEQP)]], bufs[b], gsem[b])

        def s_copy(c, b):
            return pltpu.make_async_copy(
                bufs[b], out_hbm.at[base + c], ssem[b])

        g_copy(0, 0).start()

        def blk_body(i, carry):
            for b in range(NBUF):
                c = i * NBUF + b
                bn = (b + 1) % NBUF
                @pl.when(c >= NBUF - 1)
                def _():
                    s_copy(c - NBUF + 1, bn).wait()

                @pl.when(c + 1 < ROWS_W)
                def _():
                    g_copy(c + 1, bn).start()

                g_copy(c, b).wait()
                s_copy(c, b).start()
            return carry

        lax.fori_loop(0, ROWS_W // NBUF, blk_body, 0)
        for c in range(ROWS_W - NBUF + 1, ROWS_W):
            s_copy(c, c % NBUF).wait()

    return emb


_emb = _make_emb()


def _copy_body_first(shard_ref, out_ref):
    out_ref[...] = shard_ref[:, :SEQ, :]


def _copy_body(full_ref, shard_ref, out_ref):
    del full_ref
    out_ref[...] = shard_ref[:, :SEQ, :]


def _copy_body_last(full_ref, shard_ref, p0, p1, p2, out_ref):
    del full_ref, p0, p1, p2
    out_ref[...] = shard_ref[:, :SEQ, :]


_NBLK = BS // BLK


def _out_spec(k):
    return pl.BlockSpec((BLK, SEQ, D), lambda i, _k=k: (_k * _NBLK + i, 0, 0))


_SHARD_SPEC = pl.BlockSpec((BLK, SEQP, D), lambda i: (i, 0, 0))
_OUT_SHAPE = jax.ShapeDtypeStruct((BATCH, SEQ, D), jnp.float32)
_ANY = pl.BlockSpec(memory_space=pl.ANY)

_tc_first = pl.pallas_call(
    _copy_body_first, grid=(_NBLK,), in_specs=[_SHARD_SPEC],
    out_specs=_out_spec(0), out_shape=_OUT_SHAPE)
_tc_mid = [
    pl.pallas_call(
        _copy_body, grid=(_NBLK,), in_specs=[_ANY, _SHARD_SPEC],
        out_specs=_out_spec(k), out_shape=_OUT_SHAPE,
        input_output_aliases={0: 0})
    for k in (1, 2)
]
_tc_last = pl.pallas_call(
    _copy_body_last, grid=(_NBLK,), in_specs=[_ANY, _SHARD_SPEC, _ANY, _ANY, _ANY],
    out_specs=_out_spec(NSHARD - 1), out_shape=_OUT_SHAPE,
    input_output_aliases={0: 0})


def kernel(x, table):
    xi = x.astype(jnp.int32)
    xp = jnp.pad(xi, ((0, 0), (0, SEQP - SEQ)), mode="edge")
    shards = [
        _emb(table, xp[k * BS : (k + 1) * BS].reshape(-1))
        for k in range(NSHARD)
    ]
    out = _tc_first(shards[0])
    out = _tc_mid[0](out, shards[1])
    out = _tc_mid[1](out, shards[2])
    out = _tc_last(out, shards[3], shards[0], shards[1], shards[2])
    return out


# speedup vs baseline: 1.0154x; 1.0154x over previous
import functools

import jax
import jax.numpy as jnp
from jax import lax
from jax.experimental import pallas as pl
from jax.experimental.pallas import tpu as pltpu
from jax.experimental.pallas import tpu_sc as plsc

BATCH = 4096
SEQ = 50
SEQP = 56
SEQF = 48
D = 512
NC = 2
NS = 16
NW = NC * NS
ROWS_W = BATCH // NW
NBUF = 4


def _make_emb():
    mesh = plsc.VectorSubcoreMesh(core_axis_name="c", subcore_axis_name="s")

    @functools.partial(
        pl.kernel,
        mesh=mesh,
        out_type=jax.ShapeDtypeStruct((BATCH, SEQP, D), jnp.float32),
        scratch_types=[
            pltpu.VMEM((ROWS_W * SEQP,), jnp.int32),
        ]
        + [pltpu.VMEM((SEQP, D), jnp.float32) for _ in range(NBUF)]
        + [pltpu.SemaphoreType.DMA for _ in range(2 * NBUF)],
    )
    def emb(table_hbm, idx_hbm, out_hbm, idx_v, *bufs_and_sems):
        bufs = bufs_and_sems[:NBUF]
        gsem = bufs_and_sems[NBUF : 2 * NBUF]
        ssem = bufs_and_sems[2 * NBUF : 3 * NBUF]

        wid = lax.axis_index("s") * NC + lax.axis_index("c")
        base = wid * ROWS_W
        pltpu.sync_copy(idx_hbm.at[pl.ds(base * SEQP, ROWS_W * SEQP)], idx_v)

        def g_copy(c, b):
            return pltpu.make_async_copy(
                table_hbm.at[idx_v.at[pl.ds(c * SEQP, SEQP)]], bufs[b], gsem[b])

        def s_copies(c, b):
            return (
                pltpu.make_async_copy(
                    bufs[b].at[pl.ds(0, SEQF)],
                    out_hbm.at[base + c, pl.ds(0, SEQF)],
                    ssem[b]),
                pltpu.make_async_copy(
                    bufs[b].at[pl.ds(SEQF, SEQ - SEQF)],
                    out_hbm.at[base + c, pl.ds(SEQF, SEQ - SEQF)],
                    ssem[b]),
            )

        def s_start(c, b):
            for cp in s_copies(c, b):
                cp.start()

        def s_wait(c, b):
            for cp in s_copies(c, b):
                cp.wait()

        g_copy(0, 0).start()

        def blk(i, carry):
            for b in range(NBUF):
                c = i * NBUF + b
                bn = (b + 1) % NBUF
                @pl.when(c >= NBUF - 1)
                def _():
                    s_wait(c - NBUF + 1, bn)

                @pl.when(c + 1 < ROWS_W)
                def _():
                    g_copy(c + 1, bn).start()

                g_copy(c, b).wait()
                s_start(c, b)
            return carry

        lax.fori_loop(0, ROWS_W // NBUF, blk, 0)
        for c in range(ROWS_W - NBUF + 1, ROWS_W):
            s_wait(c, c % NBUF)

    return emb


_emb = _make_emb()

BLK = 32


def _slice_body(in_ref, out_ref):
    out_ref[...] = in_ref[:, :SEQ, :]


_tc_slice = pl.pallas_call(
    _slice_body,
    grid=(BATCH // BLK,),
    in_specs=[pl.BlockSpec((BLK, SEQP, D), lambda i: (i, 0, 0))],
    out_specs=pl.BlockSpec((BLK, SEQ, D), lambda i: (i, 0, 0)),
    out_shape=jax.ShapeDtypeStruct((BATCH, SEQ, D), jnp.float32),
)


def kernel(x, table):
    xi = x.astype(jnp.int32)
    xp = jnp.pad(xi, ((0, 0), (0, SEQP - SEQ)), mode="edge")
    out = _emb(table, xp.reshape(-1))
    return _tc_slice(out)


# speedup vs baseline: 1.5586x; 1.5349x over previous
import functools

import jax
import jax.numpy as jnp
from jax import lax
from jax.experimental import pallas as pl
from jax.experimental.pallas import tpu as pltpu
from jax.experimental.pallas import tpu_sc as plsc

BATCH = 4096
SEQ = 50
SEQP = 56
SEQF = 48
D = 512
NC = 2
NS = 16
NW = NC * NS
ROWS_W = BATCH // NW
NBUF = 4


def _make_emb():
    mesh = plsc.VectorSubcoreMesh(core_axis_name="c", subcore_axis_name="s")

    @functools.partial(
        pl.kernel,
        mesh=mesh,
        out_type=jax.ShapeDtypeStruct((BATCH, SEQP, D), jnp.float32),
        scratch_types=[
            pltpu.VMEM((ROWS_W * SEQP,), jnp.int32),
        ]
        + [pltpu.VMEM((SEQP, D), jnp.float32) for _ in range(NBUF)]
        + [pltpu.SemaphoreType.DMA for _ in range(2 * NBUF)],
    )
    def emb(table_hbm, idx_hbm, out_hbm, idx_v, *bufs_and_sems):
        bufs = bufs_and_sems[:NBUF]
        gsem = bufs_and_sems[NBUF : 2 * NBUF]
        ssem = bufs_and_sems[2 * NBUF : 3 * NBUF]

        wid = lax.axis_index("s") * NC + lax.axis_index("c")
        base = wid * ROWS_W
        pltpu.sync_copy(idx_hbm.at[pl.ds(base * SEQP, ROWS_W * SEQP)], idx_v)

        def g_copy(c, b):
            return pltpu.make_async_copy(
                table_hbm.at[idx_v.at[pl.ds(c * SEQP, SEQP)]], bufs[b], gsem[b])

        def s_copies(c, b):
            return (
                pltpu.make_async_copy(
                    bufs[b].at[pl.ds(0, SEQF)],
                    out_hbm.at[base + c, pl.ds(0, SEQF)],
                    ssem[b]),
                pltpu.make_async_copy(
                    bufs[b].at[pl.ds(SEQF, SEQ - SEQF)],
                    out_hbm.at[base + c, pl.ds(SEQF, SEQ - SEQF)],
                    ssem[b]),
            )

        def s_start(c, b):
            for cp in s_copies(c, b):
                cp.start()

        def s_wait(c, b):
            for cp in s_copies(c, b):
                cp.wait()

        g_copy(0, 0).start()

        def blk(i, carry):
            for b in range(NBUF):
                c = i * NBUF + b
                bn = (b + 1) % NBUF
                @pl.when(c >= NBUF - 1)
                def _():
                    s_wait(c - NBUF + 1, bn)

                @pl.when(c + 1 < ROWS_W)
                def _():
                    g_copy(c + 1, bn).start()

                g_copy(c, b).wait()
                s_start(c, b)
            return carry

        lax.fori_loop(0, ROWS_W // NBUF, blk, 0)
        for c in range(ROWS_W - NBUF + 1, ROWS_W):
            s_wait(c, c % NBUF)

    return emb


_emb = _make_emb()


def kernel(x, table):
    xi = x.astype(jnp.int32)
    xp = jnp.pad(xi, ((0, 0), (0, SEQP - SEQ)), mode="edge")
    out = _emb(table, xp.reshape(-1))
    return out[:, :SEQ, :]
